# baseline (speedup 1.0000x reference)
import functools

import jax
import jax.numpy as jnp
from jax import lax
from jax.experimental import pallas as pl
from jax.experimental.pallas import tpu as pltpu
from jax.experimental.pallas import tpu_sc as plsc

BATCH = 16384
SEQ = 200
EMB = 64
N = BATCH * SEQ
NC, NS = 2, 16
NW = NC * NS
PER_W = N // NW
BROW = 4
CHUNK = BROW * SEQ
NCHUNK = PER_W // CHUNK
NPAIR = NCHUNK // 2
GROUPS = CHUNK // 16

_MESH = plsc.VectorSubcoreMesh(core_axis_name="c", subcore_axis_name="s")


@functools.partial(
    pl.kernel,
    out_type=jax.ShapeDtypeStruct((BATCH, SEQ, EMB), jnp.float32),
    mesh=_MESH,
    scratch_types=[
        pltpu.VMEM((5 * EMB,), jnp.float32),
        pltpu.VMEM((CHUNK,), jnp.int32),
        pltpu.VMEM((CHUNK,), jnp.int32),
        pltpu.VMEM((BROW, SEQ, EMB), jnp.float32),
        pltpu.VMEM((BROW, SEQ, EMB), jnp.float32),
        pltpu.SemaphoreType.DMA,
        pltpu.SemaphoreType.DMA,
    ],
    compiler_params=pltpu.CompilerParams(
        use_tc_tiling_on_sc=False, needs_layout_passes=False),
)
def _emb_lookup(x_hbm, tab_hbm, out_hbm, tab_v, idx_a, idx_b, rows_a, rows_b,
                sem_a, sem_b):
    wid = lax.axis_index("s") * NC + lax.axis_index("c")
    base = wid * PER_W

    pltpu.sync_copy(tab_hbm, tab_v)

    def build(idx_v, rows_v):
        @plsc.parallel_loop(0, GROUPS, step=1, unroll=1)
        def group(k):
            idxv = idx_v[pl.ds(k * 16, 16)]
            for r in range(16):
                i = k * 16 + r
                a = i // SEQ
                b = i % SEQ
                tbase = idxv[r] * EMB
                for j in range(EMB // 16):
                    rows_v[a, b, pl.ds(16 * j, 16)] = (
                        tab_v[pl.ds(tbase + 16 * j, 16)])

    def pair(t, carry):
        off0 = base + (2 * t) * CHUNK
        brow0 = off0 // SEQ

        pltpu.sync_copy(x_hbm.at[pl.ds(off0, CHUNK)], idx_a)

        @pl.when(t > 0)
        def _():
            pltpu.make_async_copy(
                rows_a, out_hbm.at[pl.ds(brow0 - 2 * BROW, BROW)],
                sem_a).wait()

        build(idx_a, rows_a)
        pltpu.make_async_copy(
            rows_a, out_hbm.at[pl.ds(brow0, BROW)], sem_a).start()

        off1 = off0 + CHUNK
        brow1 = brow0 + BROW
        pltpu.sync_copy(x_hbm.at[pl.ds(off1, CHUNK)], idx_b)

        @pl.when(t > 0)
        def _():
            pltpu.make_async_copy(
                rows_b, out_hbm.at[pl.ds(brow1 - 2 * BROW, BROW)],
                sem_b).wait()

        build(idx_b, rows_b)
        pltpu.make_async_copy(
            rows_b, out_hbm.at[pl.ds(brow1, BROW)], sem_b).start()
        return carry

    lax.fori_loop(0, NPAIR, pair, 0)

    lastb = (base + (NCHUNK - 2) * CHUNK) // SEQ
    pltpu.make_async_copy(
        rows_a, out_hbm.at[pl.ds(lastb, BROW)], sem_a).wait()
    pltpu.make_async_copy(
        rows_b, out_hbm.at[pl.ds(lastb + BROW, BROW)], sem_b).wait()


def kernel(x, connectivity_embedding):
    x1d = x.reshape(-1)
    tab1d = connectivity_embedding.reshape(-1)
    return _emb_lookup(x1d, tab1d)


# speedup vs baseline: 1.3152x; 1.3152x over previous
import functools

import jax
import jax.numpy as jnp
from jax import lax
from jax.experimental import pallas as pl
from jax.experimental.pallas import tpu as pltpu
from jax.experimental.pallas import tpu_sc as plsc

BATCH = 16384
SEQ = 200
EMB = 64
N = BATCH * SEQ
NC, NS = 2, 16
NW = NC * NS
PER_W = N // NW
BROW = 2
CHUNK = BROW * SEQ
NCHUNK = PER_W // CHUNK
NPAIR = NCHUNK // 2
GROUPS = CHUNK // 16

_MESH = plsc.VectorSubcoreMesh(core_axis_name="c", subcore_axis_name="s")


@functools.partial(
    pl.kernel,
    out_type=jax.ShapeDtypeStruct((BATCH, SEQ, EMB), jnp.float32),
    mesh=_MESH,
    scratch_types=[
        pltpu.VMEM((5 * EMB,), jnp.float32),
        pltpu.VMEM((CHUNK,), jnp.int32),
        pltpu.VMEM((CHUNK,), jnp.int32),
        pltpu.VMEM((BROW, SEQ, EMB), jnp.float32),
        pltpu.VMEM((BROW, SEQ, EMB), jnp.float32),
        pltpu.SemaphoreType.DMA,
        pltpu.SemaphoreType.DMA,
    ],
)
def _emb_lookup(x_hbm, tab_hbm, out_hbm, tab_v, idx_a, idx_b, rows_a, rows_b,
                sem_a, sem_b):
    wid = lax.axis_index("s") * NC + lax.axis_index("c")
    base = wid * PER_W

    pltpu.sync_copy(tab_hbm, tab_v)

    def build(idx_v, rows_v):
        @plsc.parallel_loop(0, GROUPS, step=1, unroll=1)
        def group(k):
            idxv = idx_v[pl.ds(k * 16, 16)]
            for r in range(16):
                i = k * 16 + r
                a = i // SEQ
                b = i % SEQ
                tbase = idxv[r] * EMB
                for j in range(EMB // 16):
                    rows_v[a, b, pl.ds(16 * j, 16)] = (
                        tab_v[pl.ds(tbase + 16 * j, 16)])

    def pair(t, carry):
        off0 = base + (2 * t) * CHUNK
        brow0 = off0 // SEQ

        pltpu.sync_copy(x_hbm.at[pl.ds(off0, CHUNK)], idx_a)

        @pl.when(t > 0)
        def _():
            pltpu.make_async_copy(
                rows_a, out_hbm.at[pl.ds(brow0 - 2 * BROW, BROW)],
                sem_a).wait()

        build(idx_a, rows_a)
        pltpu.make_async_copy(
            rows_a, out_hbm.at[pl.ds(brow0, BROW)], sem_a).start()

        off1 = off0 + CHUNK
        brow1 = brow0 + BROW
        pltpu.sync_copy(x_hbm.at[pl.ds(off1, CHUNK)], idx_b)

        @pl.when(t > 0)
        def _():
            pltpu.make_async_copy(
                rows_b, out_hbm.at[pl.ds(brow1 - 2 * BROW, BROW)],
                sem_b).wait()

        build(idx_b, rows_b)
        pltpu.make_async_copy(
            rows_b, out_hbm.at[pl.ds(brow1, BROW)], sem_b).start()
        return carry

    lax.fori_loop(0, NPAIR, pair, 0)

    lastb = (base + (NCHUNK - 2) * CHUNK) // SEQ
    pltpu.make_async_copy(
        rows_a, out_hbm.at[pl.ds(lastb, BROW)], sem_a).wait()
    pltpu.make_async_copy(
        rows_b, out_hbm.at[pl.ds(lastb + BROW, BROW)], sem_b).wait()


def kernel(x, connectivity_embedding):
    x1d = x.reshape(-1)
    tab1d = connectivity_embedding.reshape(-1)
    return _emb_lookup(x1d, tab1d)
